# Initial kernel scaffold; baseline (speedup 1.0000x reference)
import jax
import jax.numpy as jnp
from jax.experimental import pallas as pl


def kernel(image_indices, rays, depth_mask, rot_dict, trans_dict):
    raise NotImplementedError("write your pallas kernel here")


# speedup vs baseline: 9.8072x; 9.8072x over previous
import functools

import jax
import jax.numpy as jnp
from jax import lax
from jax.experimental import pallas as pl
from jax.experimental.pallas import tpu as pltpu
from jax.experimental.pallas import tpu_sc as plsc

_NF_PAD = 1024


def _table_body(rot_ref, trans_ref, out_ref):
    wx = rot_ref[0:1, :]
    wy = rot_ref[1:2, :]
    wz = rot_ref[2:3, :]
    t2 = wx * wx + wy * wy + wz * wz
    theta = jnp.sqrt(jnp.maximum(t2, 1e-24))
    small = t2 < 1e-8
    a = jnp.where(small, 1.0 - t2 / 6.0, jnp.sin(theta) / theta)
    b = jnp.where(small, 0.5 - t2 / 24.0,
                  (1.0 - jnp.cos(theta)) / jnp.maximum(t2, 1e-24))
    axy = b * wx * wy
    axz = b * wx * wz
    ayz = b * wy * wz
    r00 = 1.0 - b * (wy * wy + wz * wz)
    r11 = 1.0 - b * (wx * wx + wz * wz)
    r22 = 1.0 - b * (wx * wx + wy * wy)
    r01 = axy - a * wz
    r10 = axy + a * wz
    r02 = axz + a * wy
    r20 = axz - a * wy
    r12 = ayz - a * wx
    r21 = ayz + a * wx
    z = jnp.zeros_like(wx)
    out_ref[...] = jnp.concatenate(
        [r00, r01, r02, r10, r11, r12, r20, r21, r22,
         trans_ref[0:1, :], trans_ref[1:2, :], trans_ref[2:3, :],
         z, z, z, z], axis=0)


def _make_table(rot_p, trans_p):
    return pl.pallas_call(
        _table_body,
        out_shape=jax.ShapeDtypeStruct((16, _NF_PAD), jnp.float32),
    )(rot_p, trans_p)


def _sc_apply(table_flat, idx, mask, rays_flat, n_rays):
    info = plsc.get_sparse_core_info()
    nc, ns = info.num_cores, info.num_subcores
    nw = nc * ns
    ch = n_rays // nw
    iters = ch // 16
    mesh = plsc.VectorSubcoreMesh(core_axis_name="c", subcore_axis_name="s")

    @functools.partial(
        pl.kernel,
        out_type=jax.ShapeDtypeStruct((n_rays * 8,), jnp.float32),
        mesh=mesh,
        scratch_types=[
            pltpu.VMEM((16 * _NF_PAD,), jnp.float32),
            pltpu.VMEM((ch,), jnp.int32),
            pltpu.VMEM((ch,), jnp.int32),
            pltpu.VMEM((ch * 8,), jnp.float32),
            pltpu.VMEM((ch * 8,), jnp.float32),
        ],
        compiler_params=pltpu.CompilerParams(needs_layout_passes=False),
    )
    def body(tab_hbm, idx_hbm, mask_hbm, rays_hbm, out_hbm,
             tab_v, idx_v, mask_v, rays_v, out_v):
        wid = lax.axis_index("s") * nc + lax.axis_index("c")
        base = wid * ch
        pltpu.sync_copy(tab_hbm, tab_v)
        pltpu.sync_copy(idx_hbm.at[pl.ds(base, ch)], idx_v)
        pltpu.sync_copy(mask_hbm.at[pl.ds(base, ch)], mask_v)
        pltpu.sync_copy(rays_hbm.at[pl.ds(base * 8, ch * 8)], rays_v)

        iota = lax.iota(jnp.int32, 16)
        iota8 = iota * 8
        fzero = jnp.zeros((16,), jnp.float32)
        fone = jnp.ones((16,), jnp.float32)

        def step(i, carry):
            s = i * 16
            fidx = idx_v[pl.ds(s, 16)]
            m = mask_v[pl.ds(s, 16)] == 1
            rbase = iota8 + s * 8
            ray = [plsc.load_gather(rays_v, [rbase + c]) for c in range(8)]
            g = [plsc.load_gather(tab_v, [fidx + (c * _NF_PAD)])
                 for c in range(12)]
            r00 = jnp.where(m, g[0], fone)
            r01 = jnp.where(m, g[1], fzero)
            r02 = jnp.where(m, g[2], fzero)
            r10 = jnp.where(m, g[3], fzero)
            r11 = jnp.where(m, g[4], fone)
            r12 = jnp.where(m, g[5], fzero)
            r20 = jnp.where(m, g[6], fzero)
            r21 = jnp.where(m, g[7], fzero)
            r22 = jnp.where(m, g[8], fone)
            t0 = jnp.where(m, g[9], fzero)
            t1 = jnp.where(m, g[10], fzero)
            t2 = jnp.where(m, g[11], fzero)
            d0, d1, d2 = ray[3], ray[4], ray[5]
            keep = (iota + (base + s)) >= 6
            outs = [
                ray[0] + t0,
                ray[1] + t1,
                ray[2] + t2,
                r00 * d0 + r01 * d1 + r02 * d2,
                r10 * d0 + r11 * d1 + r12 * d2,
                r20 * d0 + r21 * d1 + r22 * d2,
                jnp.where(keep, ray[6], fzero),
                jnp.where(keep, ray[7], fzero),
            ]
            for c, o in enumerate(outs):
                plsc.store_scatter(out_v, [rbase + c], o)
            return carry

        lax.fori_loop(0, iters, step, 0)
        pltpu.sync_copy(out_v, out_hbm.at[pl.ds(base * 8, ch * 8)])

    return body(table_flat, idx, mask, rays_flat)


def kernel(image_indices, rays, depth_mask, rot_dict, trans_dict):
    n = rays.shape[0]
    nf = rot_dict.shape[0]
    idx = image_indices.astype(jnp.int32)
    mask = depth_mask.reshape(n).astype(jnp.int32)
    rot_p = jnp.zeros((8, _NF_PAD), jnp.float32).at[:3, :nf].set(
        rot_dict.astype(jnp.float32).T)
    trans_p = jnp.zeros((8, _NF_PAD), jnp.float32).at[:3, :nf].set(
        trans_dict.astype(jnp.float32).T)
    table = _make_table(rot_p, trans_p)
    out_flat = _sc_apply(table.reshape(16 * _NF_PAD), idx, mask,
                         rays.astype(jnp.float32).reshape(n * 8), n)
    return out_flat.reshape(n, 8)
